# Initial kernel scaffold; baseline (speedup 1.0000x reference)
import jax
import jax.numpy as jnp
from jax.experimental import pallas as pl


def kernel(ori, embeds):
    raise NotImplementedError("write your pallas kernel here")


# speedup vs baseline: 2.2563x; 2.2563x over previous
import functools
import jax
import jax.numpy as jnp
from jax import lax
from jax.experimental import pallas as pl
from jax.experimental.pallas import tpu as pltpu
from jax.experimental.pallas import tpu_sc as plsc

N_EMBED = 100000
N_LAYER = 16
CH = 64
D = N_LAYER * CH
B_TOT = 16384
NC, NS, LANES = 2, 16, 16
NW = NC * NS
BPW = B_TOT // NW
CB = 32
NCHUNK = BPW // CB

_mesh = plsc.VectorSubcoreMesh(core_axis_name="c", subcore_axis_name="s")


@functools.partial(
    pl.kernel,
    out_type=jax.ShapeDtypeStruct((B_TOT, D), jnp.float32),
    mesh=_mesh,
    scratch_types=[
        pltpu.VMEM((BPW,), jnp.float32),
        pltpu.VMEM((CB,), jnp.int32),
        pltpu.VMEM((CB,), jnp.int32),
        pltpu.VMEM((CB,), jnp.float32),
        pltpu.VMEM((CB,), jnp.float32),
        pltpu.VMEM((CB, D), jnp.float32),
        pltpu.VMEM((CB, D), jnp.float32),
        pltpu.SemaphoreType.DMA,
        pltpu.SemaphoreType.DMA,
    ],
    compiler_params=pltpu.CompilerParams(needs_layout_passes=False),
)
def _embed_lookup(ori_hbm, table_hbm, out_hbm,
                  ori_v, idxl_v, idxr_v, wl_v, wr_v, bufl, bufr, seml, semr):
    wid = lax.axis_index("s") * NC + lax.axis_index("c")
    base = wid * BPW
    pltpu.sync_copy(ori_hbm.at[pl.ds(base, BPW)], ori_v)

    def chunk_body(k, carry):
        for g in range(CB // LANES):
            o = ori_v[pl.ds(k * CB + g * LANES, LANES)]
            s = (o + 1.0) * 0.5 * float(N_EMBED)
            i0 = s.astype(jnp.int32)
            f = s - i0.astype(jnp.float32)
            il = jnp.where(i0 >= N_EMBED, i0 - N_EMBED, i0)
            i1 = i0 + 1
            ir = jnp.where(i1 >= N_EMBED, i1 - N_EMBED, i1)
            sl = pl.ds(g * LANES, LANES)
            idxl_v[sl] = il
            idxr_v[sl] = ir
            wl_v[sl] = 1.0 - f
            wr_v[sl] = f

        cl = pltpu.async_copy(table_hbm.at[idxl_v], bufl, seml)
        cr = pltpu.async_copy(table_hbm.at[idxr_v], bufr, semr)
        cl.wait()
        cr.wait()

        def row_body(j, carry2):
            jv = jnp.zeros((LANES,), jnp.int32) + j
            wl = plsc.load_gather(wl_v, [jv])
            wr = plsc.load_gather(wr_v, [jv])

            def col_body(v, carry3):
                sl = pl.ds(v * LANES, LANES)
                left = bufl[j, sl]
                right = bufr[j, sl]
                bufl[j, sl] = left * wl + right * wr
                return carry3

            lax.fori_loop(0, D // LANES, col_body, 0, unroll=8)
            return carry2

        lax.fori_loop(0, CB, row_body, 0)
        pltpu.sync_copy(bufl, out_hbm.at[pl.ds(base + k * CB, CB)])
        return carry

    lax.fori_loop(0, NCHUNK, chunk_body, 0)


def kernel(ori, embeds):
    table = embeds.reshape(N_EMBED, D)
    out = _embed_lookup(ori, table)
    return out.reshape(B_TOT, N_LAYER, CH)


# speedup vs baseline: 2.4579x; 1.0893x over previous
import functools
import jax
import jax.numpy as jnp
from jax import lax
from jax.experimental import pallas as pl
from jax.experimental.pallas import tpu as pltpu
from jax.experimental.pallas import tpu_sc as plsc

N_EMBED = 100000
N_LAYER = 16
CH = 64
D = N_LAYER * CH
B_TOT = 16384
NC, NS, LANES = 2, 16, 16
NW = NC * NS
BPW = B_TOT // NW
CB = 16
NCHUNK = BPW // CB
NBUF = 2

_mesh = plsc.VectorSubcoreMesh(core_axis_name="c", subcore_axis_name="s")


@functools.partial(
    pl.kernel,
    out_type=jax.ShapeDtypeStruct((B_TOT, D), jnp.float32),
    mesh=_mesh,
    scratch_types=[
        pltpu.VMEM((BPW,), jnp.float32),
        [pltpu.VMEM((CB,), jnp.int32) for _ in range(NBUF)],
        [pltpu.VMEM((CB,), jnp.int32) for _ in range(NBUF)],
        [pltpu.VMEM((CB,), jnp.float32) for _ in range(NBUF)],
        [pltpu.VMEM((CB,), jnp.float32) for _ in range(NBUF)],
        [pltpu.VMEM((CB, D), jnp.float32) for _ in range(NBUF)],
        [pltpu.VMEM((CB, D), jnp.float32) for _ in range(NBUF)],
        [pltpu.VMEM((CB, D), jnp.float32) for _ in range(NBUF)],
        [pltpu.SemaphoreType.DMA for _ in range(NBUF)],
        [pltpu.SemaphoreType.DMA for _ in range(NBUF)],
    ],
    compiler_params=pltpu.CompilerParams(needs_layout_passes=False),
)
def _embed_lookup(ori_hbm, table_hbm, out_hbm,
                  ori_v, idxl, idxr, wl_v, wr_v, bufl, bufr, obuf, gsem, osem):
    wid = lax.axis_index("s") * NC + lax.axis_index("c")
    base = wid * BPW
    pltpu.sync_copy(ori_hbm.at[pl.ds(base, BPW)], ori_v)

    def stage_indices(k, b):
        for g in range(CB // LANES):
            o = ori_v[pl.ds(k * CB + g * LANES, LANES)]
            s = (o + 1.0) * 0.5 * float(N_EMBED)
            i0 = s.astype(jnp.int32)
            f = s - i0.astype(jnp.float32)
            il = jnp.where(i0 >= N_EMBED, i0 - N_EMBED, i0)
            i1 = i0 + 1
            ir = jnp.where(i1 >= N_EMBED, i1 - N_EMBED, i1)
            sl = pl.ds(g * LANES, LANES)
            idxl[b][sl] = il
            idxr[b][sl] = ir
            wl_v[b][sl] = 1.0 - f
            wr_v[b][sl] = f

    def start_gathers(b):
        pltpu.async_copy(table_hbm.at[idxl[b]], bufl[b], gsem[b])
        pltpu.async_copy(table_hbm.at[idxr[b]], bufr[b], gsem[b])

    def wait_gathers(b):
        pltpu.make_async_copy(table_hbm.at[idxl[b]], bufl[b], gsem[b]).wait()
        pltpu.make_async_copy(table_hbm.at[idxr[b]], bufr[b], gsem[b]).wait()

    def interp_chunk(b):
        def row_body(j, carry):
            jv = jnp.zeros((LANES,), jnp.int32) + j
            wl = plsc.load_gather(wl_v[b], [jv])
            wr = plsc.load_gather(wr_v[b], [jv])

            def col_body(v, carry2):
                sl = pl.ds(v * LANES, LANES)
                obuf[b][j, sl] = bufl[b][j, sl] * wl + bufr[b][j, sl] * wr
                return carry2

            lax.fori_loop(0, D // LANES, col_body, 0, unroll=8)
            return carry

        lax.fori_loop(0, CB, row_body, 0)

    stage_indices(0, 0)
    start_gathers(0)

    def outer(kk, carry):
        for b in range(NBUF):
            k = kk + b

            @pl.when(k + 1 < NCHUNK)
            def _():
                stage_indices(k + 1, 1 - b)
                start_gathers(1 - b)

            wait_gathers(b)

            @pl.when(k >= NBUF)
            def _():
                pltpu.make_async_copy(
                    obuf[b], out_hbm.at[pl.ds(base, CB)], osem[b]).wait()

            interp_chunk(b)
            pltpu.async_copy(
                obuf[b], out_hbm.at[pl.ds(base + k * CB, CB)], osem[b])
        return carry

    lax.fori_loop(0, NCHUNK // NBUF, lambda i, c: outer(i * NBUF, c), 0)

    for b in range(NBUF):
        pltpu.make_async_copy(
            obuf[b], out_hbm.at[pl.ds(base, CB)], osem[b]).wait()


def kernel(ori, embeds):
    table = embeds.reshape(N_EMBED, D)
    out = _embed_lookup(ori, table)
    return out.reshape(B_TOT, N_LAYER, CH)
